# baseline (speedup 1.0000x reference)
import functools

import jax
import jax.numpy as jnp
from jax import lax
from jax.experimental import pallas as pl
from jax.experimental.pallas import tpu as pltpu
from jax.experimental.pallas import tpu_sc as plsc

D = 64
LANES = 16
NC, NS = 2, 16
NW = NC * NS
B, SEQ = 4096, 200
N = B * SEQ
B_PER_W = B // NW
W = SEQ
NCHUNK = B_PER_W
STAGE_C = 201
POS_PAD = 256
_SPLITS = ((0, 128), (128, 72))


def _comb_body(wt_ref, pos_ref, out_ref):
    out_ref[...] = wt_ref[...][:, None, :] + pos_ref[...][None, :, :]


def _build_comb(word_type_table, pos_table_padded):
    out3 = pl.pallas_call(
        _comb_body,
        out_shape=jax.ShapeDtypeStruct((8, POS_PAD, D), jnp.float32),
    )(word_type_table, pos_table_padded)
    return out3.reshape(8 * POS_PAD, D)


def _sc_body(seq_hbm, wt_hbm, pos_hbm, word_hbm, comb_hbm, out_hbm,
             seq_v, wt_v, pos_v, cidx_v, rows_w, rows_c, stage, sems):
    wid = lax.axis_index("s") * NC + lax.axis_index("c")
    b0 = wid * B_PER_W

    def load_and_fire(b, s):
        base = b * SEQ
        pltpu.sync_copy(seq_hbm.at[pl.ds(base, W)], seq_v[s])
        pltpu.sync_copy(wt_hbm.at[pl.ds(base, W)], wt_v[s])
        pltpu.sync_copy(pos_hbm.at[pl.ds(base, W)], pos_v[s])
        offs = [t * LANES for t in range(W // LANES)] + [W - LANES]
        for off in offs:
            sl = pl.ds(off, LANES)
            cidx_v[s][sl] = wt_v[s][sl] * POS_PAD + pos_v[s][sl]
        for off, n in _SPLITS:
            sl = pl.ds(off, n)
            pltpu.async_copy(word_hbm.at[seq_v[s].at[sl]], rows_w[s].at[sl],
                             sems[s])
            pltpu.async_copy(comb_hbm.at[cidx_v[s].at[sl]], rows_c[s].at[sl],
                             sems[s])

    def drain(s):
        pltpu.make_async_copy(word_hbm.at[pl.ds(0, W)], rows_w[s],
                              sems[s]).wait()
        pltpu.make_async_copy(comb_hbm.at[pl.ds(0, W)], rows_c[s],
                              sems[s]).wait()

    def process_and_store(b, s):
        lane = lax.iota(jnp.int32, LANES)

        @pl.loop(0, W)
        def _row(r):
            idx_r = jnp.full((LANES,), 0, jnp.int32) + r
            for c in range(D // LANES):
                sl2 = pl.ds(c * LANES, LANES)
                val = rows_w[s][r, sl2] + rows_c[s][r, sl2]
                plsc.store_scatter(stage[s], [lane + (c * LANES), idx_r], val)

        pltpu.sync_copy(stage[s].at[:, pl.ds(0, SEQ)], out_hbm.at[b])

    load_and_fire(b0, 0)

    @pl.loop(0, NCHUNK // 2)
    def _pair(i):
        g = i * 2
        load_and_fire(b0 + g + 1, 1)
        drain(0)
        process_and_store(b0 + g, 0)

        @pl.when(g + 2 < NCHUNK)
        def _():
            load_and_fire(b0 + g + 2, 0)

        drain(1)
        process_and_store(b0 + g + 1, 1)


@functools.partial(
    pl.kernel,
    out_type=jax.ShapeDtypeStruct((B, D, SEQ), jnp.float32),
    mesh=plsc.VectorSubcoreMesh(core_axis_name="c", subcore_axis_name="s"),
    compiler_params=pltpu.CompilerParams(use_tc_tiling_on_sc=False,
                                        needs_layout_passes=False),
    scratch_types=[
        pltpu.VMEM((W,), jnp.int32), pltpu.VMEM((W,), jnp.int32),
        pltpu.VMEM((W,), jnp.int32), pltpu.VMEM((W,), jnp.int32),
        pltpu.VMEM((W,), jnp.int32), pltpu.VMEM((W,), jnp.int32),
        pltpu.VMEM((W,), jnp.int32), pltpu.VMEM((W,), jnp.int32),
        pltpu.VMEM((W, D), jnp.float32), pltpu.VMEM((W, D), jnp.float32),
        pltpu.VMEM((W, D), jnp.float32), pltpu.VMEM((W, D), jnp.float32),
        pltpu.VMEM((D, STAGE_C), jnp.float32),
        pltpu.VMEM((D, STAGE_C), jnp.float32),
        pltpu.SemaphoreType.DMA, pltpu.SemaphoreType.DMA,
    ],
)
def _sc_lookup(seq_hbm, wt_hbm, pos_hbm, word_hbm, comb_hbm, out_hbm,
               seq0, seq1, wt0, wt1, pos0, pos1, cidx0, cidx1,
               roww0, roww1, rowc0, rowc1, st0, st1, sem0, sem1):
    _sc_body(seq_hbm, wt_hbm, pos_hbm, word_hbm, comb_hbm, out_hbm,
             (seq0, seq1), (wt0, wt1), (pos0, pos1), (cidx0, cidx1),
             (roww0, roww1), (rowc0, rowc1), (st0, st1), (sem0, sem1))


@jax.jit
def kernel(sequence, wtype, pos_enc, src_word_table, word_type_table,
           src_pos_table):
    seq = sequence.reshape(-1).astype(jnp.int32)
    wt = wtype.reshape(-1).astype(jnp.int32)
    pos = pos_enc.reshape(-1).astype(jnp.int32)
    pos_padded = jnp.pad(src_pos_table,
                         ((0, POS_PAD - src_pos_table.shape[0]), (0, 0)))
    comb = _build_comb(word_type_table, pos_padded)
    outT = _sc_lookup(seq, wt, pos, src_word_table, comb)
    return jnp.transpose(outT, (0, 2, 1))


# speedup vs baseline: 1.2325x; 1.2325x over previous
import functools

import numpy as np

import jax
import jax.numpy as jnp
from jax import lax
from jax.experimental import pallas as pl
from jax.experimental.pallas import tpu as pltpu
from jax.experimental.pallas import tpu_sc as plsc

D = 64
LANES = 16
NC, NS = 2, 16
NW = NC * NS
B, SEQ = 4096, 200
N = B * SEQ
PER_W = N // NW
W = 256
NCHUNK = PER_W // W
GATHER = 128
NG = W // GATHER
POS_PAD = 256


def _comb_body(wt_ref, pos_ref, out_ref):
    out_ref[...] = wt_ref[...][:, None, :] + pos_ref[...][None, :, :]


def _build_comb(word_type_table, pos_table_padded):
    out3 = pl.pallas_call(
        _comb_body,
        out_shape=jax.ShapeDtypeStruct((8, POS_PAD, D), jnp.float32),
    )(word_type_table, pos_table_padded)
    return out3.reshape(8 * POS_PAD, D)


def _sc_body(seq_hbm, wt_hbm, pos_hbm, word_hbm, comb_hbm, out_hbm,
             seq_v, wt_v, pos_v, cidx_v, rows_w, rows_c, stage, sems):
    wid = lax.axis_index("s") * NC + lax.axis_index("c")
    base0 = wid * PER_W

    def load_and_fire(base, s):
        pltpu.sync_copy(seq_hbm.at[pl.ds(base, W)], seq_v[s])
        pltpu.sync_copy(wt_hbm.at[pl.ds(base, W)], wt_v[s])
        pltpu.sync_copy(pos_hbm.at[pl.ds(base, W)], pos_v[s])
        for t in range(W // LANES):
            sl = pl.ds(t * LANES, LANES)
            cidx_v[s][sl] = wt_v[s][sl] * POS_PAD + pos_v[s][sl]
        for j in range(NG):
            sl = pl.ds(j * GATHER, GATHER)
            pltpu.async_copy(word_hbm.at[seq_v[s].at[sl]], rows_w[s].at[sl],
                             sems[s])
            pltpu.async_copy(comb_hbm.at[cidx_v[s].at[sl]], rows_c[s].at[sl],
                             sems[s])

    def drain(s):
        pltpu.make_async_copy(word_hbm.at[pl.ds(0, W)], rows_w[s],
                              sems[s]).wait()
        pltpu.make_async_copy(comb_hbm.at[pl.ds(0, W)], rows_c[s],
                              sems[s]).wait()

    def process_and_store(base, s):
        @pl.loop(0, W)
        def _row(r):
            for c in range(D // 32):
                sl32 = pl.ds(c * 32, 32)
                aw, bw = plsc.unpack(rows_w[s][r, sl32],
                                     format=plsc.PackFormat.INTERLEAVED,
                                     preferred_element_type=jnp.float32)
                ac, bc = plsc.unpack(rows_c[s][r, sl32],
                                     format=plsc.PackFormat.INTERLEAVED,
                                     preferred_element_type=jnp.float32)
                stage[s][r, pl.ds(c * 32, LANES)] = aw + ac
                stage[s][r, pl.ds(c * 32 + LANES, LANES)] = bw + bc

        pltpu.sync_copy(stage[s], out_hbm.at[pl.ds(base, W)])

    load_and_fire(base0, 0)

    @pl.loop(0, NCHUNK // 2)
    def _pair(i):
        g = i * 2
        load_and_fire(base0 + (g + 1) * W, 1)
        drain(0)
        process_and_store(base0 + g * W, 0)

        @pl.when(g + 2 < NCHUNK)
        def _():
            load_and_fire(base0 + (g + 2) * W, 0)

        drain(1)
        process_and_store(base0 + (g + 1) * W, 1)


@functools.partial(
    pl.kernel,
    out_type=jax.ShapeDtypeStruct((N, D), jnp.float32),
    mesh=plsc.VectorSubcoreMesh(core_axis_name="c", subcore_axis_name="s"),
    compiler_params=pltpu.CompilerParams(use_tc_tiling_on_sc=False,
                                        needs_layout_passes=False),
    scratch_types=[
        pltpu.VMEM((W,), jnp.int32), pltpu.VMEM((W,), jnp.int32),
        pltpu.VMEM((W,), jnp.int32), pltpu.VMEM((W,), jnp.int32),
        pltpu.VMEM((W,), jnp.int32), pltpu.VMEM((W,), jnp.int32),
        pltpu.VMEM((W,), jnp.int32), pltpu.VMEM((W,), jnp.int32),
        pltpu.VMEM((W, D), jnp.bfloat16), pltpu.VMEM((W, D), jnp.bfloat16),
        pltpu.VMEM((W, D), jnp.bfloat16), pltpu.VMEM((W, D), jnp.bfloat16),
        pltpu.VMEM((W, D), jnp.float32), pltpu.VMEM((W, D), jnp.float32),
        pltpu.SemaphoreType.DMA, pltpu.SemaphoreType.DMA,
    ],
)
def _sc_lookup(seq_hbm, wt_hbm, pos_hbm, word_hbm, comb_hbm, out_hbm,
               seq0, seq1, wt0, wt1, pos0, pos1, cidx0, cidx1,
               roww0, roww1, rowc0, rowc1, st0, st1, sem0, sem1):
    _sc_body(seq_hbm, wt_hbm, pos_hbm, word_hbm, comb_hbm, out_hbm,
             (seq0, seq1), (wt0, wt1), (pos0, pos1), (cidx0, cidx1),
             (roww0, roww1), (rowc0, rowc1), (st0, st1), (sem0, sem1))


_PERM = np.zeros(D, np.int32)
for _c in range(D // 32):
    for _k in range(16):
        _PERM[_c * 32 + 2 * _k] = _c * 32 + _k
        _PERM[_c * 32 + 2 * _k + 1] = _c * 32 + 16 + _k


@jax.jit
def kernel(sequence, wtype, pos_enc, src_word_table, word_type_table,
           src_pos_table):
    seq = sequence.reshape(-1).astype(jnp.int32)
    wt = wtype.reshape(-1).astype(jnp.int32)
    pos = pos_enc.reshape(-1).astype(jnp.int32)
    pos_padded = jnp.pad(src_pos_table,
                         ((0, POS_PAD - src_pos_table.shape[0]), (0, 0)))
    comb = _build_comb(word_type_table, pos_padded)
    word16 = src_word_table[:, _PERM].astype(jnp.bfloat16)
    comb16 = comb[:, _PERM].astype(jnp.bfloat16)
    out = _sc_lookup(seq, wt, pos, word16, comb16)
    return out.reshape(B, SEQ, D)


# speedup vs baseline: 1.6052x; 1.3024x over previous
import functools

import jax
import jax.numpy as jnp
from jax import lax
from jax.experimental import pallas as pl
from jax.experimental.pallas import tpu as pltpu
from jax.experimental.pallas import tpu_sc as plsc

D = 64
LANES = 16
NC, NS = 2, 16
NW = NC * NS
B, SEQ = 4096, 200
N = B * SEQ
PER_W = N // NW
W = 256
NCHUNK = PER_W // W
GATHER = 128
NG = W // GATHER
POS_PAD = 256


def _comb_body(wt_ref, pos_ref, out_ref):
    out_ref[...] = wt_ref[...][:, None, :] + pos_ref[...][None, :, :]


def _build_comb(word_type_table, pos_table_padded):
    out3 = pl.pallas_call(
        _comb_body,
        out_shape=jax.ShapeDtypeStruct((8, POS_PAD, D), jnp.float32),
    )(word_type_table, pos_table_padded)
    return out3.reshape(8 * POS_PAD, D)


def _sc_body(seq_hbm, wt_hbm, pos_hbm, word_hbm, comb_hbm, out_hbm,
             seq_v, wt_v, pos_v, cidx_v, rows_w, rows_c, sems):
    wid = lax.axis_index("s") * NC + lax.axis_index("c")
    base0 = wid * PER_W

    def load_and_fire(base, s):
        pltpu.sync_copy(seq_hbm.at[pl.ds(base, W)], seq_v[s])
        pltpu.sync_copy(wt_hbm.at[pl.ds(base, W)], wt_v[s])
        pltpu.sync_copy(pos_hbm.at[pl.ds(base, W)], pos_v[s])
        for t in range(W // LANES):
            sl = pl.ds(t * LANES, LANES)
            cidx_v[s][sl] = wt_v[s][sl] * POS_PAD + pos_v[s][sl]
        for j in range(NG):
            sl = pl.ds(j * GATHER, GATHER)
            pltpu.async_copy(word_hbm.at[seq_v[s].at[sl]], rows_w[s].at[sl],
                             sems[s])
            pltpu.async_copy(comb_hbm.at[cidx_v[s].at[sl]], rows_c[s].at[sl],
                             sems[s])

    def drain(s):
        pltpu.make_async_copy(word_hbm.at[pl.ds(0, W)], rows_w[s],
                              sems[s]).wait()
        pltpu.make_async_copy(comb_hbm.at[pl.ds(0, W)], rows_c[s],
                              sems[s]).wait()

    def process_and_store(base, s):
        @pl.loop(0, W)
        def _row(r):
            for c in range(D // LANES):
                sl2 = pl.ds(c * LANES, LANES)
                plsc.addupdate(rows_w[s].at[r, sl2], rows_c[s][r, sl2])

        pltpu.sync_copy(rows_w[s], out_hbm.at[pl.ds(base, W)])

    load_and_fire(base0, 0)

    @pl.loop(0, NCHUNK // 2)
    def _pair(i):
        g = i * 2
        load_and_fire(base0 + (g + 1) * W, 1)
        drain(0)
        process_and_store(base0 + g * W, 0)

        @pl.when(g + 2 < NCHUNK)
        def _():
            load_and_fire(base0 + (g + 2) * W, 0)

        drain(1)
        process_and_store(base0 + (g + 1) * W, 1)


@functools.partial(
    pl.kernel,
    out_type=jax.ShapeDtypeStruct((N, D), jnp.float32),
    mesh=plsc.VectorSubcoreMesh(core_axis_name="c", subcore_axis_name="s"),
    compiler_params=pltpu.CompilerParams(use_tc_tiling_on_sc=False),
    scratch_types=[
        pltpu.VMEM((W,), jnp.int32), pltpu.VMEM((W,), jnp.int32),
        pltpu.VMEM((W,), jnp.int32), pltpu.VMEM((W,), jnp.int32),
        pltpu.VMEM((W,), jnp.int32), pltpu.VMEM((W,), jnp.int32),
        pltpu.VMEM((W,), jnp.int32), pltpu.VMEM((W,), jnp.int32),
        pltpu.VMEM((W, D), jnp.float32), pltpu.VMEM((W, D), jnp.float32),
        pltpu.VMEM((W, D), jnp.float32), pltpu.VMEM((W, D), jnp.float32),
        pltpu.SemaphoreType.DMA, pltpu.SemaphoreType.DMA,
    ],
)
def _sc_lookup(seq_hbm, wt_hbm, pos_hbm, word_hbm, comb_hbm, out_hbm,
               seq0, seq1, wt0, wt1, pos0, pos1, cidx0, cidx1,
               roww0, roww1, rowc0, rowc1, sem0, sem1):
    _sc_body(seq_hbm, wt_hbm, pos_hbm, word_hbm, comb_hbm, out_hbm,
             (seq0, seq1), (wt0, wt1), (pos0, pos1), (cidx0, cidx1),
             (roww0, roww1), (rowc0, rowc1), (sem0, sem1))


@jax.jit
def kernel(sequence, wtype, pos_enc, src_word_table, word_type_table,
           src_pos_table):
    seq = sequence.reshape(-1).astype(jnp.int32)
    wt = wtype.reshape(-1).astype(jnp.int32)
    pos = pos_enc.reshape(-1).astype(jnp.int32)
    pos_padded = jnp.pad(src_pos_table,
                         ((0, POS_PAD - src_pos_table.shape[0]), (0, 0)))
    comb = _build_comb(word_type_table, pos_padded)
    out = _sc_lookup(seq, wt, pos, src_word_table, comb)
    return out.reshape(B, SEQ, D)


# speedup vs baseline: 1.6528x; 1.0296x over previous
import functools

import jax
import jax.numpy as jnp
from jax import lax
from jax.experimental import pallas as pl
from jax.experimental.pallas import tpu as pltpu
from jax.experimental.pallas import tpu_sc as plsc

D = 64
LANES = 16
NC, NS = 2, 16
NW = NC * NS
B, SEQ = 4096, 200
N = B * SEQ
PER_W = N // NW
W = 320
NCHUNK = PER_W // W
GATHER = 128
_SPLITS = ((0, 128), (128, 128), (256, 64))
POS_PAD = 256


def _comb_body(wt_ref, pos_ref, out_ref):
    out_ref[...] = wt_ref[...][:, None, :] + pos_ref[...][None, :, :]


def _build_comb(word_type_table, pos_table_padded):
    out3 = pl.pallas_call(
        _comb_body,
        out_shape=jax.ShapeDtypeStruct((8, POS_PAD, D), jnp.float32),
    )(word_type_table, pos_table_padded)
    return out3.reshape(8 * POS_PAD, D)


def _sc_body(seq_hbm, wt_hbm, pos_hbm, word_hbm, comb_hbm, out_hbm,
             seq_v, wt_v, pos_v, cidx_v, rows_w, rows_c, sems):
    wid = lax.axis_index("s") * NC + lax.axis_index("c")
    base0 = wid * PER_W

    def load_and_fire(base, s):
        pltpu.sync_copy(seq_hbm.at[pl.ds(base, W)], seq_v[s])
        pltpu.sync_copy(wt_hbm.at[pl.ds(base, W)], wt_v[s])
        pltpu.sync_copy(pos_hbm.at[pl.ds(base, W)], pos_v[s])
        for t in range(W // LANES):
            sl = pl.ds(t * LANES, LANES)
            cidx_v[s][sl] = wt_v[s][sl] * POS_PAD + pos_v[s][sl]
        for off, n in _SPLITS:
            sl = pl.ds(off, n)
            pltpu.async_copy(word_hbm.at[seq_v[s].at[sl]], rows_w[s].at[sl],
                             sems[s])
            pltpu.async_copy(comb_hbm.at[cidx_v[s].at[sl]], rows_c[s].at[sl],
                             sems[s])

    def drain(s):
        pltpu.make_async_copy(word_hbm.at[pl.ds(0, W)], rows_w[s],
                              sems[s]).wait()
        pltpu.make_async_copy(comb_hbm.at[pl.ds(0, W)], rows_c[s],
                              sems[s]).wait()

    def process_and_store(base, s):
        @pl.loop(0, W)
        def _row(r):
            for c in range(D // LANES):
                sl2 = pl.ds(c * LANES, LANES)
                plsc.addupdate(rows_w[s].at[r, sl2], rows_c[s][r, sl2])

        pltpu.sync_copy(rows_w[s], out_hbm.at[pl.ds(base, W)])

    load_and_fire(base0, 0)

    @pl.loop(0, NCHUNK // 2)
    def _pair(i):
        g = i * 2
        load_and_fire(base0 + (g + 1) * W, 1)
        drain(0)
        process_and_store(base0 + g * W, 0)

        @pl.when(g + 2 < NCHUNK)
        def _():
            load_and_fire(base0 + (g + 2) * W, 0)

        drain(1)
        process_and_store(base0 + (g + 1) * W, 1)


@functools.partial(
    pl.kernel,
    out_type=jax.ShapeDtypeStruct((N, D), jnp.float32),
    mesh=plsc.VectorSubcoreMesh(core_axis_name="c", subcore_axis_name="s"),
    compiler_params=pltpu.CompilerParams(use_tc_tiling_on_sc=False),
    scratch_types=[
        pltpu.VMEM((W,), jnp.int32), pltpu.VMEM((W,), jnp.int32),
        pltpu.VMEM((W,), jnp.int32), pltpu.VMEM((W,), jnp.int32),
        pltpu.VMEM((W,), jnp.int32), pltpu.VMEM((W,), jnp.int32),
        pltpu.VMEM((W,), jnp.int32), pltpu.VMEM((W,), jnp.int32),
        pltpu.VMEM((W, D), jnp.float32), pltpu.VMEM((W, D), jnp.float32),
        pltpu.VMEM((W, D), jnp.float32), pltpu.VMEM((W, D), jnp.float32),
        pltpu.SemaphoreType.DMA, pltpu.SemaphoreType.DMA,
    ],
)
def _sc_lookup(seq_hbm, wt_hbm, pos_hbm, word_hbm, comb_hbm, out_hbm,
               seq0, seq1, wt0, wt1, pos0, pos1, cidx0, cidx1,
               roww0, roww1, rowc0, rowc1, sem0, sem1):
    _sc_body(seq_hbm, wt_hbm, pos_hbm, word_hbm, comb_hbm, out_hbm,
             (seq0, seq1), (wt0, wt1), (pos0, pos1), (cidx0, cidx1),
             (roww0, roww1), (rowc0, rowc1), (sem0, sem1))


@jax.jit
def kernel(sequence, wtype, pos_enc, src_word_table, word_type_table,
           src_pos_table):
    seq = sequence.reshape(-1).astype(jnp.int32)
    wt = wtype.reshape(-1).astype(jnp.int32)
    pos = pos_enc.reshape(-1).astype(jnp.int32)
    pos_padded = jnp.pad(src_pos_table,
                         ((0, POS_PAD - src_pos_table.shape[0]), (0, 0)))
    comb = _build_comb(word_type_table, pos_padded)
    out = _sc_lookup(seq, wt, pos, src_word_table, comb)
    return out.reshape(B, SEQ, D)


# speedup vs baseline: 1.6785x; 1.0156x over previous
import functools

import jax
import jax.numpy as jnp
from jax import lax
from jax.experimental import pallas as pl
from jax.experimental.pallas import tpu as pltpu
from jax.experimental.pallas import tpu_sc as plsc

D = 64
LANES = 16
NC, NS = 2, 16
NW = NC * NS
B, SEQ = 4096, 200
N = B * SEQ
PER_W = N // NW
W = 400
NCHUNK = PER_W // W
GATHER = 128
_SPLITS = ((0, 128), (128, 128), (256, 128), (384, 16))
POS_PAD = 256


def _comb_body(wt_ref, pos_ref, out_ref):
    out_ref[...] = wt_ref[...][:, None, :] + pos_ref[...][None, :, :]


def _build_comb(word_type_table, pos_table_padded):
    out3 = pl.pallas_call(
        _comb_body,
        out_shape=jax.ShapeDtypeStruct((8, POS_PAD, D), jnp.float32),
    )(word_type_table, pos_table_padded)
    return out3.reshape(8 * POS_PAD, D)


def _sc_body(seq_hbm, wt_hbm, pos_hbm, word_hbm, comb_hbm, out_hbm,
             seq_v, wt_v, pos_v, cidx_v, rows_w, rows_c, sems):
    wid = lax.axis_index("s") * NC + lax.axis_index("c")
    base0 = wid * PER_W

    def load_and_fire(base, s):
        pltpu.sync_copy(seq_hbm.at[pl.ds(base, W)], seq_v[s])
        pltpu.sync_copy(wt_hbm.at[pl.ds(base, W)], wt_v[s])
        pltpu.sync_copy(pos_hbm.at[pl.ds(base, W)], pos_v[s])
        for t in range(W // LANES):
            sl = pl.ds(t * LANES, LANES)
            cidx_v[s][sl] = wt_v[s][sl] * POS_PAD + pos_v[s][sl]
        for off, n in _SPLITS:
            sl = pl.ds(off, n)
            pltpu.async_copy(word_hbm.at[seq_v[s].at[sl]], rows_w[s].at[sl],
                             sems[s])
            pltpu.async_copy(comb_hbm.at[cidx_v[s].at[sl]], rows_c[s].at[sl],
                             sems[s])

    def drain(s):
        pltpu.make_async_copy(word_hbm.at[pl.ds(0, W)], rows_w[s],
                              sems[s]).wait()
        pltpu.make_async_copy(comb_hbm.at[pl.ds(0, W)], rows_c[s],
                              sems[s]).wait()

    def process_and_store(base, s):
        @pl.loop(0, W)
        def _row(r):
            for c in range(D // LANES):
                sl2 = pl.ds(c * LANES, LANES)
                plsc.addupdate(rows_w[s].at[r, sl2], rows_c[s][r, sl2])

        pltpu.sync_copy(rows_w[s], out_hbm.at[pl.ds(base, W)])

    load_and_fire(base0, 0)

    @pl.loop(0, NCHUNK // 2)
    def _pair(i):
        g = i * 2
        load_and_fire(base0 + (g + 1) * W, 1)
        drain(0)
        process_and_store(base0 + g * W, 0)

        @pl.when(g + 2 < NCHUNK)
        def _():
            load_and_fire(base0 + (g + 2) * W, 0)

        drain(1)
        process_and_store(base0 + (g + 1) * W, 1)


@functools.partial(
    pl.kernel,
    out_type=jax.ShapeDtypeStruct((N, D), jnp.float32),
    mesh=plsc.VectorSubcoreMesh(core_axis_name="c", subcore_axis_name="s"),
    compiler_params=pltpu.CompilerParams(use_tc_tiling_on_sc=False),
    scratch_types=[
        pltpu.VMEM((W,), jnp.int32), pltpu.VMEM((W,), jnp.int32),
        pltpu.VMEM((W,), jnp.int32), pltpu.VMEM((W,), jnp.int32),
        pltpu.VMEM((W,), jnp.int32), pltpu.VMEM((W,), jnp.int32),
        pltpu.VMEM((W,), jnp.int32), pltpu.VMEM((W,), jnp.int32),
        pltpu.VMEM((W, D), jnp.float32), pltpu.VMEM((W, D), jnp.float32),
        pltpu.VMEM((W, D), jnp.float32), pltpu.VMEM((W, D), jnp.float32),
        pltpu.SemaphoreType.DMA, pltpu.SemaphoreType.DMA,
    ],
)
def _sc_lookup(seq_hbm, wt_hbm, pos_hbm, word_hbm, comb_hbm, out_hbm,
               seq0, seq1, wt0, wt1, pos0, pos1, cidx0, cidx1,
               roww0, roww1, rowc0, rowc1, sem0, sem1):
    _sc_body(seq_hbm, wt_hbm, pos_hbm, word_hbm, comb_hbm, out_hbm,
             (seq0, seq1), (wt0, wt1), (pos0, pos1), (cidx0, cidx1),
             (roww0, roww1), (rowc0, rowc1), (sem0, sem1))


@jax.jit
def kernel(sequence, wtype, pos_enc, src_word_table, word_type_table,
           src_pos_table):
    seq = sequence.reshape(-1).astype(jnp.int32)
    wt = wtype.reshape(-1).astype(jnp.int32)
    pos = pos_enc.reshape(-1).astype(jnp.int32)
    pos_padded = jnp.pad(src_pos_table,
                         ((0, POS_PAD - src_pos_table.shape[0]), (0, 0)))
    comb = _build_comb(word_type_table, pos_padded)
    out = _sc_lookup(seq, wt, pos, src_word_table, comb)
    return out.reshape(B, SEQ, D)
